# Initial kernel scaffold; baseline (speedup 1.0000x reference)
import functools

import jax
import jax.numpy as jnp
from jax import lax
from jax.experimental import pallas as pl
from jax.experimental.pallas import tpu as pltpu
from jax.experimental.pallas import tpu_sc as plsc

B, C, H, W = 16, 384, 56, 56
HALF = C // 2
ROW = H * W
ROWS = B * C
CHUNK = 16
NCHUNK = HALF // CHUNK


def _body(a_hbm, b_hbm, out_hbm, idx_v, buf0, buf1, sem0, sem1):
    cid = lax.axis_index("c")
    sid = lax.axis_index("s")
    wid = cid * 16 + sid
    batch = wid % 16
    parity = wid // 16
    base = batch * C
    lane = lax.iota(jnp.int32, (16,))

    for k in range(NCHUNK):
        idx_v[k, :] = base + 2 * (k * CHUNK) + 2 * lane + parity

    def run(src_hbm, out_base):
        for k in range(NCHUNK):
            buf = buf0 if k % 2 == 0 else buf1
            sem = sem0 if k % 2 == 0 else sem1
            pltpu.async_copy(src_hbm.at[idx_v.at[k]], buf, sem).wait()
            pltpu.sync_copy(buf, out_hbm.at[pl.ds(out_base + k * CHUNK, CHUNK)])

    @pl.when(parity == 0)
    def _():
        run(a_hbm, base)

    @pl.when(parity == 1)
    def _():
        run(b_hbm, base + HALF)


@jax.jit
def _frozen_adder(a2, b2):
    mesh = plsc.VectorSubcoreMesh(core_axis_name="c", subcore_axis_name="s")
    return pl.kernel(
        _body,
        out_type=jax.ShapeDtypeStruct((ROWS, ROW), jnp.float32),
        mesh=mesh,
        scratch_types=[
            pltpu.VMEM((NCHUNK, CHUNK), jnp.int32),
            pltpu.VMEM((CHUNK, ROW), jnp.float32),
            pltpu.VMEM((CHUNK, ROW), jnp.float32),
            pltpu.SemaphoreType.DMA,
            pltpu.SemaphoreType.DMA,
        ],
    )(a2, b2)


def kernel(input_a, input_b):
    a2 = input_a.reshape(ROWS, ROW)
    b2 = input_b.reshape(ROWS, ROW)
    out = _frozen_adder(a2, b2)
    return out.reshape(B, C, H, W)


# speedup vs baseline: 1.3262x; 1.3262x over previous
import functools

import jax
import jax.numpy as jnp
from jax import lax
from jax.experimental import pallas as pl
from jax.experimental.pallas import tpu as pltpu
from jax.experimental.pallas import tpu_sc as plsc

B, C, H, W = 16, 384, 56, 56
HALF = C // 2
ROW = H * W
ROWS = B * C
CHUNK = 16
NCHUNK = HALF // CHUNK


def _body(a_hbm, b_hbm, out_hbm, idx_v, buf0, buf1, sem0, sem1):
    cid = lax.axis_index("c")
    sid = lax.axis_index("s")
    wid = cid * 16 + sid
    batch = wid % 16
    parity = wid // 16
    base = batch * C
    lane = lax.iota(jnp.int32, 16)

    for k in range(NCHUNK):
        idx_v[k, :] = base + 2 * (k * CHUNK) + 2 * lane + parity

    def run(src_hbm, out_base):
        for k in range(NCHUNK):
            buf = buf0 if k % 2 == 0 else buf1
            sem = sem0 if k % 2 == 0 else sem1
            pltpu.async_copy(src_hbm.at[idx_v.at[k]], buf, sem).wait()
            pltpu.sync_copy(buf, out_hbm.at[pl.ds(out_base + k * CHUNK, CHUNK)])

    @pl.when(parity == 0)
    def _():
        run(a_hbm, base)

    @pl.when(parity == 1)
    def _():
        run(b_hbm, base + HALF)


@jax.jit
def _frozen_adder(a2, b2):
    mesh = plsc.VectorSubcoreMesh(core_axis_name="c", subcore_axis_name="s")
    return pl.kernel(
        _body,
        out_type=jax.ShapeDtypeStruct((ROWS, ROW), jnp.float32),
        mesh=mesh,
        scratch_types=[
            pltpu.VMEM((NCHUNK, CHUNK), jnp.int32),
            pltpu.VMEM((CHUNK, ROW), jnp.float32),
            pltpu.VMEM((CHUNK, ROW), jnp.float32),
            pltpu.SemaphoreType.DMA,
            pltpu.SemaphoreType.DMA,
        ],
        compiler_params=pltpu.CompilerParams(use_tc_tiling_on_sc=False),
    )(a2, b2)


def kernel(input_a, input_b):
    a2 = input_a.reshape(ROWS, ROW)
    b2 = input_b.reshape(ROWS, ROW)
    out = _frozen_adder(a2, b2)
    return out.reshape(B, C, H, W)


# speedup vs baseline: 1.3376x; 1.0086x over previous
import functools

import jax
import jax.numpy as jnp
from jax import lax
from jax.experimental import pallas as pl
from jax.experimental.pallas import tpu as pltpu
from jax.experimental.pallas import tpu_sc as plsc

B, C, H, W = 16, 384, 56, 56
HALF = C // 2
ROW = H * W
ROWS = B * C
CHUNK = 8
NCHUNK = HALF // CHUNK
NBUF = 4


def _body(a_hbm, b_hbm, out_hbm, idx_v, bufs,
          gsem0, gsem1, gsem2, gsem3, wsem0, wsem1, wsem2, wsem3):
    gsems = (gsem0, gsem1, gsem2, gsem3)
    wsems = (wsem0, wsem1, wsem2, wsem3)
    cid = lax.axis_index("c")
    sid = lax.axis_index("s")
    wid = cid * 16 + sid
    batch = wid % 16
    parity = wid // 16
    base = batch * C
    lane = lax.iota(jnp.int32, 16)

    for j in range(HALF // 16):
        idx_v[pl.ds(j * 16, 16)] = base + 2 * (j * 16) + 2 * lane + parity

    def run(src_hbm, out_base):
        def gather_copy(k):
            b = k % NBUF
            return pltpu.make_async_copy(
                src_hbm.at[idx_v.at[pl.ds(k * CHUNK, CHUNK)]],
                bufs.at[b], gsems[b])

        def write_copy(k):
            b = k % NBUF
            return pltpu.make_async_copy(
                bufs.at[b], out_hbm.at[pl.ds(out_base + k * CHUNK, CHUNK)],
                wsems[b])

        for j in range(NBUF):
            gather_copy(j).start()
        for k in range(NCHUNK):
            gather_copy(k).wait()
            wcp = write_copy(k)
            wcp.start()
            wcp.wait()
            if k + NBUF < NCHUNK:
                gather_copy(k + NBUF).start()

    @pl.when(parity == 0)
    def _():
        run(a_hbm, base)

    @pl.when(parity == 1)
    def _():
        run(b_hbm, base + HALF)


@jax.jit
def _frozen_adder(a2, b2):
    mesh = plsc.VectorSubcoreMesh(core_axis_name="c", subcore_axis_name="s")
    return pl.kernel(
        _body,
        out_type=jax.ShapeDtypeStruct((ROWS, ROW), jnp.float32),
        mesh=mesh,
        scratch_types=[
            pltpu.VMEM((HALF,), jnp.int32),
            pltpu.VMEM((NBUF, CHUNK, ROW), jnp.float32),
            pltpu.SemaphoreType.DMA,
            pltpu.SemaphoreType.DMA,
            pltpu.SemaphoreType.DMA,
            pltpu.SemaphoreType.DMA,
            pltpu.SemaphoreType.DMA,
            pltpu.SemaphoreType.DMA,
            pltpu.SemaphoreType.DMA,
            pltpu.SemaphoreType.DMA,
        ],
        compiler_params=pltpu.CompilerParams(use_tc_tiling_on_sc=False),
    )(a2, b2)


def kernel(input_a, input_b):
    a2 = input_a.reshape(ROWS, ROW)
    b2 = input_b.reshape(ROWS, ROW)
    out = _frozen_adder(a2, b2)
    return out.reshape(B, C, H, W)


# speedup vs baseline: 2.7203x; 2.0337x over previous
import jax
import jax.numpy as jnp
from jax import lax
from jax.experimental import pallas as pl
from jax.experimental.pallas import tpu as pltpu
from jax.experimental.pallas import tpu_sc as plsc

B, C, H, W = 16, 384, 56, 56
HALF = C // 2
ROWS = B * C
CHUNK = 8
NCHUNK = HALF // CHUNK
NBUF = 2


def _body(a_hbm, b_hbm, out_hbm, bufs, gsem0, gsem1, wsem0, wsem1):
    gsems = (gsem0, gsem1)
    wsems = (wsem0, wsem1)
    cid = lax.axis_index("c")
    sid = lax.axis_index("s")
    wid = cid * 16 + sid
    batch = wid % 16
    parity = wid // 16
    base = batch * C

    def run(src_hbm, out_base):
        def gather_copy(k, j):
            b = k % NBUF
            src = base + 2 * (k * CHUNK + j) + parity
            return pltpu.make_async_copy(
                src_hbm.at[src], bufs.at[b, j], gsems[b])

        def write_copy(k):
            b = k % NBUF
            return pltpu.make_async_copy(
                bufs.at[b], out_hbm.at[pl.ds(out_base + k * CHUNK, CHUNK)],
                wsems[b])

        def start_gathers(k):
            for j in range(CHUNK):
                gather_copy(k, j).start()

        def wait_gathers(k):
            for j in range(CHUNK):
                gather_copy(k, j).wait()

        for j in range(NBUF):
            start_gathers(j)
        for k in range(NCHUNK):
            wait_gathers(k)
            wcp = write_copy(k)
            wcp.start()
            wcp.wait()
            if k + NBUF < NCHUNK:
                start_gathers(k + NBUF)

    @pl.when(parity == 0)
    def _():
        run(a_hbm, base)

    @pl.when(parity == 1)
    def _():
        run(b_hbm, base + HALF)


@jax.jit
def _frozen_adder(a3, b3):
    mesh = plsc.VectorSubcoreMesh(core_axis_name="c", subcore_axis_name="s")
    return pl.kernel(
        _body,
        out_type=jax.ShapeDtypeStruct((ROWS, H, W), jnp.float32),
        mesh=mesh,
        scratch_types=[
            pltpu.VMEM((NBUF, CHUNK, H, W), jnp.float32),
            pltpu.SemaphoreType.DMA,
            pltpu.SemaphoreType.DMA,
            pltpu.SemaphoreType.DMA,
            pltpu.SemaphoreType.DMA,
        ],
    )(a3, b3)


def kernel(input_a, input_b):
    a3 = input_a.reshape(ROWS, H, W)
    b3 = input_b.reshape(ROWS, H, W)
    out = _frozen_adder(a3, b3)
    return out.reshape(B, C, H, W)


# speedup vs baseline: 2.7205x; 1.0001x over previous
import jax
import jax.numpy as jnp
from jax import lax
from jax.experimental import pallas as pl
from jax.experimental.pallas import tpu as pltpu
from jax.experimental.pallas import tpu_sc as plsc

B, C, H, W = 16, 384, 56, 56
HALF = C // 2
ROWS = B * C
CHUNK = 8
NCHUNK = HALF // CHUNK
NBUF = 2


def _body(a_hbm, b_hbm, out_hbm, bufs, gsem0, gsem1, wsem0, wsem1):
    gsems = (gsem0, gsem1)
    wsems = (wsem0, wsem1)
    cid = lax.axis_index("c")
    sid = lax.axis_index("s")
    wid = cid * 16 + sid
    batch = wid % 16
    parity = wid // 16
    base = batch * C

    def run(src_hbm, out_base):
        def gather_copy(k, j):
            b = k % NBUF
            src = base + 2 * (k * CHUNK + j) + parity
            return pltpu.make_async_copy(
                src_hbm.at[src], bufs.at[b, j], gsems[b])

        def write_copy(k):
            b = k % NBUF
            return pltpu.make_async_copy(
                bufs.at[b], out_hbm.at[pl.ds(out_base + k * CHUNK, CHUNK)],
                wsems[b])

        def start_gathers(k):
            for j in range(CHUNK):
                gather_copy(k, j).start()

        def wait_gathers(k):
            for j in range(CHUNK):
                gather_copy(k, j).wait()

        for j in range(NBUF):
            start_gathers(j)
        for k in range(NCHUNK):
            wait_gathers(k)
            wcp = write_copy(k)
            wcp.start()
            wcp.wait()
            if k + NBUF < NCHUNK:
                start_gathers(k + NBUF)

    @pl.when(parity == 0)
    def _():
        run(a_hbm, base)

    @pl.when(parity == 1)
    def _():
        run(b_hbm, base + HALF)


@jax.jit
def _frozen_adder(a3, b3):
    mesh = plsc.VectorSubcoreMesh(core_axis_name="c", subcore_axis_name="s")
    return pl.kernel(
        _body,
        out_type=jax.ShapeDtypeStruct((ROWS, H, W), jnp.float32),
        mesh=mesh,
        scratch_types=[
            pltpu.VMEM((NBUF, CHUNK, H, W), jnp.float32),
            pltpu.SemaphoreType.DMA,
            pltpu.SemaphoreType.DMA,
            pltpu.SemaphoreType.DMA,
            pltpu.SemaphoreType.DMA,
        ],
        compiler_params=pltpu.CompilerParams(use_tc_tiling_on_sc=True),
    )(a3, b3)


def kernel(input_a, input_b):
    a3 = input_a.reshape(ROWS, H, W)
    b3 = input_b.reshape(ROWS, H, W)
    out = _frozen_adder(a3, b3)
    return out.reshape(B, C, H, W)


# speedup vs baseline: 5.9349x; 2.1815x over previous
import jax
import jax.numpy as jnp
from jax import lax
from jax.experimental import pallas as pl
from jax.experimental.pallas import tpu as pltpu
from jax.experimental.pallas import tpu_sc as plsc

B, C, H, W = 16, 384, 56, 56
HALF = C // 2
NPOS = B * H * W
NW = 32
PR = NPOS // NW
CR = 16
NCHUNK = PR // CR
NPAIR = NCHUNK // 2
NVEC = C // 16


def _body(a_hbm, b_hbm, out_hbm,
          abuf, bbuf, obuf, asem0, asem1, bsem0, bsem1, wsem0, wsem1):
    asems = (asem0, asem1)
    bsems = (bsem0, bsem1)
    wsems = (wsem0, wsem1)
    cid = lax.axis_index("c")
    sid = lax.axis_index("s")
    wid = cid * 16 + sid
    row0 = wid * PR

    lane2 = 2 * lax.iota(jnp.int32, 16)
    cols_a = [32 * j + lane2 for j in range(NVEC // 2)]
    cols_b = [32 * j + lane2 + 1 for j in range(NVEC // 2)]

    def in_copies(k, bsl):
        base = row0 + k * CR
        return (
            pltpu.make_async_copy(a_hbm.at[pl.ds(base, CR)], abuf.at[bsl], asems[bsl]),
            pltpu.make_async_copy(b_hbm.at[pl.ds(base, CR)], bbuf.at[bsl], bsems[bsl]),
        )

    def out_copy(k, bsl):
        base = row0 + k * CR
        return pltpu.make_async_copy(
            obuf.at[bsl], out_hbm.at[pl.ds(base, CR)], wsems[bsl])

    def compute(bsl):
        av = abuf.at[bsl]
        bv = bbuf.at[bsl]
        ov = obuf.at[bsl]

        def row_body(r, carry):
            rsplat = jnp.full((16,), r, jnp.int32)
            for j in range(NVEC // 2):
                ov[r, pl.ds(16 * j, 16)] = plsc.load_gather(av, [rsplat, cols_a[j]])
            for j in range(NVEC // 2):
                ov[r, pl.ds(HALF + 16 * j, 16)] = plsc.load_gather(bv, [rsplat, cols_b[j]])
            return carry

        lax.fori_loop(0, CR, row_body, 0)

    for cp in in_copies(0, 0):
        cp.start()

    def pair_body(i, carry):
        k0 = 2 * i
        for cp in in_copies(k0, 0):
            cp.wait()
        for cp in in_copies(k0 + 1, 1):
            cp.start()

        @pl.when(i > 0)
        def _():
            out_copy(k0 - 2, 0).wait()

        compute(0)
        out_copy(k0, 0).start()

        for cp in in_copies(k0 + 1, 1):
            cp.wait()

        @pl.when(i + 1 < NPAIR)
        def _():
            for cp in in_copies(k0 + 2, 0):
                cp.start()

        @pl.when(i > 0)
        def _():
            out_copy(k0 - 1, 1).wait()

        compute(1)
        out_copy(k0 + 1, 1).start()
        return carry

    lax.fori_loop(0, NPAIR, pair_body, 0)
    out_copy(NCHUNK - 2, 0).wait()
    out_copy(NCHUNK - 1, 1).wait()


@jax.jit
def _frozen_adder(a2, b2):
    mesh = plsc.VectorSubcoreMesh(core_axis_name="c", subcore_axis_name="s")
    return pl.kernel(
        _body,
        out_type=jax.ShapeDtypeStruct((NPOS, C), jnp.float32),
        mesh=mesh,
        scratch_types=[
            pltpu.VMEM((2, CR, C), jnp.float32),
            pltpu.VMEM((2, CR, C), jnp.float32),
            pltpu.VMEM((2, CR, C), jnp.float32),
            pltpu.SemaphoreType.DMA,
            pltpu.SemaphoreType.DMA,
            pltpu.SemaphoreType.DMA,
            pltpu.SemaphoreType.DMA,
            pltpu.SemaphoreType.DMA,
            pltpu.SemaphoreType.DMA,
        ],
        compiler_params=pltpu.CompilerParams(
            use_tc_tiling_on_sc=True, needs_layout_passes=False),
    )(a2, b2)


def kernel(input_a, input_b):
    a2 = input_a.transpose(0, 2, 3, 1).reshape(NPOS, C)
    b2 = input_b.transpose(0, 2, 3, 1).reshape(NPOS, C)
    out = _frozen_adder(a2, b2)
    return out.reshape(B, H, W, C).transpose(0, 3, 1, 2)
